# baseline (speedup 1.0000x reference)
import functools

import jax
import jax.numpy as jnp
from jax import lax
from jax.experimental import pallas as pl
from jax.experimental.pallas import tpu as pltpu
from jax.experimental.pallas import tpu_sc as plsc

F = 14.399645

NA = 100000
NP = 6400000
NM = 1000

NC, NS, L = 2, 16, 16
NW = NC * NS

NA_PAD = 131072
NP_PAD = 6553600
PPW = NP_PAD // NW
CHUNK = 2048
NCHUNK = PPW // CHUNK
ROWS_PER_CHUNK = CHUNK // 128

NM_PAD = 1024
APW = NA_PAD // NW
BROWS = APW // 128

_mesh = plsc.VectorSubcoreMesh(core_axis_name="c", subcore_axis_name="s")


@functools.partial(
    pl.kernel,
    mesh=_mesh,
    out_type=jax.ShapeDtypeStruct((NC * NA_PAD,), jnp.float32),
    compiler_params=pltpu.CompilerParams(needs_layout_passes=False),
    scratch_types=[
        pltpu.VMEM((NA,), jnp.float32),
        pltpu.VMEM((CHUNK,), jnp.int32),
        pltpu.VMEM((CHUNK,), jnp.int32),
        pltpu.VMEM((CHUNK,), jnp.float32),
        pltpu.VMEM((CHUNK,), jnp.float32),
        pltpu.VMEM((ROWS_PER_CHUNK, 128), jnp.int32),
        pltpu.VMEM((ROWS_PER_CHUNK, 128), jnp.int32),
        pltpu.VMEM((CHUNK,), jnp.float32),
        pltpu.VMEM((CHUNK,), jnp.float32),
        pltpu.VMEM_SHARED((NA_PAD,), jnp.float32),
        pltpu.SemaphoreType.DMA,
        pltpu.SemaphoreType.DMA,
        pltpu.SemaphoreType.DMA,
        pltpu.SemaphoreType.DMA,
    ],
)
def _pairs_kernel(q_hbm, dist_hbm, first_hbm, sec_hbm, zeros_hbm, part_hbm,
                  ch_v, sec0_v, sec1_v, dist0_v, dist1_v, first0_v, first1_v,
                  v0_v, v1_v, acc_sh, isem0, isem1, ssem0, ssem1):
    cid = lax.axis_index("c")
    sid = lax.axis_index("s")
    wid = cid * NS + sid
    sec_v = (sec0_v, sec1_v)
    dist_v = (dist0_v, dist1_v)
    first_v = (first0_v, first1_v)
    v_v = (v0_v, v1_v)
    isems = (isem0, isem1)
    ssems = (ssem0, ssem1)

    pltpu.sync_copy(q_hbm, ch_v)

    @pl.when(sid == 0)
    def _():
        pltpu.sync_copy(zeros_hbm, acc_sh)

    plsc.subcore_barrier()

    def start_inputs(k, b):
        off = wid * PPW + k * CHUNK
        row = wid * (PPW // 128) + k * ROWS_PER_CHUNK
        pltpu.async_copy(sec_hbm.at[pl.ds(off, CHUNK)], sec_v[b], isems[b])
        pltpu.async_copy(dist_hbm.at[pl.ds(off, CHUNK)], dist_v[b], isems[b])
        pltpu.async_copy(first_hbm.at[pl.ds(row, ROWS_PER_CHUNK), :],
                         first_v[b], isems[b])

    def wait_inputs(k, b):
        off = wid * PPW + k * CHUNK
        row = wid * (PPW // 128) + k * ROWS_PER_CHUNK
        pltpu.make_async_copy(sec_hbm.at[pl.ds(off, CHUNK)], sec_v[b],
                              isems[b]).wait()
        pltpu.make_async_copy(dist_hbm.at[pl.ds(off, CHUNK)], dist_v[b],
                              isems[b]).wait()
        pltpu.make_async_copy(first_hbm.at[pl.ds(row, ROWS_PER_CHUNK), :],
                              first_v[b], isems[b]).wait()

    def drain_scatter(b):
        for j in range(ROWS_PER_CHUNK):
            pltpu.make_async_copy(v_v[b].at[pl.ds(j * 128, 128)],
                                  acc_sh.at[first_v[b].at[j]], ssems[b]).wait()

    start_inputs(0, 0)

    def pair_body(t, carry):
        for b in range(2):
            k = 2 * t + b
            wait_inputs(k, b)

            @pl.when(k >= 1)
            def _():
                drain_scatter(1 - b)

            @pl.when(k + 1 < NCHUNK)
            def _():
                start_inputs(k + 1, 1 - b)

            sv, dv, vv = sec_v[b], dist_v[b], v_v[b]

            def inner(i, c):
                s = pl.ds(i * L, L)
                idx = sv[s]
                q = plsc.load_gather(ch_v, [idx])
                d = dv[s]
                vv[s] = (F * q) / d
                return c

            lax.fori_loop(0, CHUNK // L, inner, 0, unroll=4)

            for j in range(ROWS_PER_CHUNK):
                pltpu.async_copy(v_v[b].at[pl.ds(j * 128, 128)],
                                 acc_sh.at[first_v[b].at[j]], ssems[b],
                                 add=True)
        return carry

    lax.fori_loop(0, NCHUNK // 2, pair_body, 0)
    drain_scatter(1)

    plsc.subcore_barrier()
    seg = NA_PAD // NS
    pltpu.sync_copy(acc_sh.at[pl.ds(sid * seg, seg)],
                    part_hbm.at[pl.ds(cid * NA_PAD + sid * seg, seg)])


@functools.partial(
    pl.kernel,
    mesh=_mesh,
    out_type=(
        jax.ShapeDtypeStruct((NA_PAD,), jnp.float32),
        jax.ShapeDtypeStruct((NC * NM_PAD,), jnp.float32),
    ),
    compiler_params=pltpu.CompilerParams(needs_layout_passes=False),
    scratch_types=[
        pltpu.VMEM((APW,), jnp.float32),
        pltpu.VMEM((APW,), jnp.float32),
        pltpu.VMEM((APW,), jnp.float32),
        pltpu.VMEM((BROWS, 128), jnp.int32),
        pltpu.VMEM((APW,), jnp.float32),
        pltpu.VMEM((APW,), jnp.float32),
        pltpu.VMEM_SHARED((NM_PAD,), jnp.float32),
    ],
)
def _mol_kernel(part_hbm, q_hbm, mol_hbm, zeros_hbm, volt_hbm, bins_hbm,
                p0_v, p1_v, q_v, mol_v, v_v, c_v, bins_sh):
    cid = lax.axis_index("c")
    sid = lax.axis_index("s")
    wid = cid * NS + sid
    base = wid * APW

    @pl.when(sid == 0)
    def _():
        pltpu.sync_copy(zeros_hbm, bins_sh)

    pltpu.sync_copy(part_hbm.at[pl.ds(base, APW)], p0_v)
    pltpu.sync_copy(part_hbm.at[pl.ds(NA_PAD + base, APW)], p1_v)
    pltpu.sync_copy(q_hbm.at[pl.ds(base, APW)], q_v)
    pltpu.sync_copy(mol_hbm.at[pl.ds(wid * BROWS, BROWS), :], mol_v)
    plsc.subcore_barrier()

    def inner(i, c):
        s = pl.ds(i * L, L)
        v = p0_v[s] + p1_v[s]
        v_v[s] = v
        c_v[s] = (0.5 * v) * q_v[s]
        return c

    lax.fori_loop(0, APW // L, inner, 0)

    pltpu.sync_copy(v_v, volt_hbm.at[pl.ds(base, APW)])
    for j in range(BROWS):
        pltpu.sync_copy(c_v.at[pl.ds(j * 128, 128)],
                        bins_sh.at[mol_v.at[j]], add=True)

    plsc.subcore_barrier()

    @pl.when(sid == 0)
    def _():
        pltpu.sync_copy(bins_sh, bins_hbm.at[pl.ds(cid * NM_PAD, NM_PAD)])


def kernel(charges, pair_dist, pair_first, pair_second, mol_index, n_molecules):
    q = charges.reshape(NA)
    padp = NP_PAD - NP
    dist_p = jnp.concatenate([pair_dist, jnp.ones((padp,), jnp.float32)])
    first_p = jnp.concatenate(
        [pair_first, jnp.full((padp,), NA, jnp.int32)]).reshape(NP_PAD // 128, 128)
    sec_p = jnp.concatenate([pair_second, jnp.zeros((padp,), jnp.int32)])
    zeros_acc = jnp.zeros((NA_PAD,), jnp.float32)

    part = _pairs_kernel(q, dist_p, first_p, sec_p, zeros_acc)

    pada = NA_PAD - NA
    q_pad = jnp.concatenate([q, jnp.zeros((pada,), jnp.float32)])
    mol_p = jnp.concatenate(
        [mol_index, jnp.full((pada,), NM_PAD - 1, jnp.int32)]).reshape(NA_PAD // 128, 128)
    zeros_bins = jnp.zeros((NM_PAD,), jnp.float32)

    volt, bins = _mol_kernel(part, q_pad, mol_p, zeros_bins)

    voltage_atom = volt[:NA].reshape(NA, 1)
    coulomb_molecules = (bins[:NM] + bins[NM_PAD:NM_PAD + NM]).reshape(NM, 1)
    return (coulomb_molecules, voltage_atom)


# speedup vs baseline: 1.3554x; 1.3554x over previous
import functools

import jax
import jax.numpy as jnp
from jax import lax
from jax.experimental import pallas as pl
from jax.experimental.pallas import tpu as pltpu
from jax.experimental.pallas import tpu_sc as plsc

F = 14.399645

NA = 100000
NP = 6400000
NM = 1000

NC, NS, L = 2, 16, 16
NW = NC * NS

NA_PAD = 131072
NP_PAD = 6553600
PPW = NP_PAD // NW
CHUNK = 1024
NCHUNK = PPW // CHUNK
ROWS_PER_CHUNK = CHUNK // 128

NM_PAD = 1024
APW = NA_PAD // NW
BROWS = APW // 128

_mesh = plsc.VectorSubcoreMesh(core_axis_name="c", subcore_axis_name="s")


@functools.partial(
    pl.kernel,
    mesh=_mesh,
    out_type=jax.ShapeDtypeStruct((NC * NA_PAD,), jnp.float32),
    compiler_params=pltpu.CompilerParams(needs_layout_passes=False),
    scratch_types=[
        pltpu.VMEM((NA,), jnp.float32),
        pltpu.VMEM((CHUNK,), jnp.int32),
        pltpu.VMEM((CHUNK,), jnp.int32),
        pltpu.VMEM((CHUNK,), jnp.float32),
        pltpu.VMEM((CHUNK,), jnp.float32),
        pltpu.VMEM((ROWS_PER_CHUNK, 128), jnp.int32),
        pltpu.VMEM((ROWS_PER_CHUNK, 128), jnp.int32),
        pltpu.VMEM((ROWS_PER_CHUNK, 128), jnp.int32),
        pltpu.VMEM((ROWS_PER_CHUNK, 128), jnp.int32),
        pltpu.VMEM((CHUNK,), jnp.float32),
        pltpu.VMEM((CHUNK,), jnp.float32),
        pltpu.VMEM((CHUNK,), jnp.float32),
        pltpu.VMEM((CHUNK,), jnp.float32),
        pltpu.VMEM_SHARED((NA_PAD,), jnp.float32),
        pltpu.SemaphoreType.DMA,
        pltpu.SemaphoreType.DMA,
        pltpu.SemaphoreType.DMA,
        pltpu.SemaphoreType.DMA,
        pltpu.SemaphoreType.DMA,
        pltpu.SemaphoreType.DMA,
    ],
)
def _pairs_kernel(q_hbm, dist_hbm, first_hbm, sec_hbm, zeros_hbm, part_hbm,
                  ch_v, sec0_v, sec1_v, dist0_v, dist1_v,
                  first0_v, first1_v, first2_v, first3_v,
                  v0_v, v1_v, v2_v, v3_v, acc_sh,
                  isem0, isem1, ssem0, ssem1, ssem2, ssem3):
    cid = lax.axis_index("c")
    sid = lax.axis_index("s")
    wid = cid * NS + sid
    sec_v = (sec0_v, sec1_v)
    dist_v = (dist0_v, dist1_v)
    first_v = (first0_v, first1_v, first2_v, first3_v)
    v_v = (v0_v, v1_v, v2_v, v3_v)
    isems = (isem0, isem1)
    ssems = (ssem0, ssem1, ssem2, ssem3)

    pltpu.sync_copy(q_hbm, ch_v)

    @pl.when(sid == 0)
    def _():
        pltpu.sync_copy(zeros_hbm, acc_sh)

    plsc.subcore_barrier()

    def start_inputs(k, b2, b4):
        off = wid * PPW + k * CHUNK
        row = wid * (PPW // 128) + k * ROWS_PER_CHUNK
        pltpu.async_copy(sec_hbm.at[pl.ds(off, CHUNK)], sec_v[b2], isems[b2])
        pltpu.async_copy(dist_hbm.at[pl.ds(off, CHUNK)], dist_v[b2], isems[b2])
        pltpu.async_copy(first_hbm.at[pl.ds(row, ROWS_PER_CHUNK), :],
                         first_v[b4], isems[b2])

    def wait_inputs(k, b2, b4):
        off = wid * PPW + k * CHUNK
        row = wid * (PPW // 128) + k * ROWS_PER_CHUNK
        pltpu.make_async_copy(sec_hbm.at[pl.ds(off, CHUNK)], sec_v[b2],
                              isems[b2]).wait()
        pltpu.make_async_copy(dist_hbm.at[pl.ds(off, CHUNK)], dist_v[b2],
                              isems[b2]).wait()
        pltpu.make_async_copy(first_hbm.at[pl.ds(row, ROWS_PER_CHUNK), :],
                              first_v[b4], isems[b2]).wait()

    def drain_scatter(b4):
        for j in range(ROWS_PER_CHUNK):
            pltpu.make_async_copy(v_v[b4].at[pl.ds(j * 128, 128)],
                                  acc_sh.at[first_v[b4].at[j]],
                                  ssems[b4]).wait()

    start_inputs(0, 0, 0)

    def pair_body(t, carry):
        for u in range(4):
            k = 4 * t + u
            b2, b4 = u % 2, u
            wait_inputs(k, b2, b4)

            @pl.when(k >= 3)
            def _():
                drain_scatter((u + 1) % 4)

            @pl.when(k + 1 < NCHUNK)
            def _():
                start_inputs(k + 1, (u + 1) % 2, (u + 1) % 4)

            sv, dv, vv = sec_v[b2], dist_v[b2], v_v[b4]

            def inner(i, c):
                s = pl.ds(i * L, L)
                idx = sv[s]
                q = plsc.load_gather(ch_v, [idx])
                d = dv[s]
                vv[s] = (F * q) / d
                return c

            lax.fori_loop(0, CHUNK // L, inner, 0, unroll=4)

            for j in range(ROWS_PER_CHUNK):
                pltpu.async_copy(v_v[b4].at[pl.ds(j * 128, 128)],
                                 acc_sh.at[first_v[b4].at[j]], ssems[b4],
                                 add=True)
        return carry

    lax.fori_loop(0, NCHUNK // 4, pair_body, 0)
    drain_scatter(1)
    drain_scatter(2)
    drain_scatter(3)

    plsc.subcore_barrier()
    seg = NA_PAD // NS
    pltpu.sync_copy(acc_sh.at[pl.ds(sid * seg, seg)],
                    part_hbm.at[pl.ds(cid * NA_PAD + sid * seg, seg)])


@functools.partial(
    pl.kernel,
    mesh=_mesh,
    out_type=(
        jax.ShapeDtypeStruct((NA_PAD,), jnp.float32),
        jax.ShapeDtypeStruct((NC * NM_PAD,), jnp.float32),
    ),
    compiler_params=pltpu.CompilerParams(needs_layout_passes=False),
    scratch_types=[
        pltpu.VMEM((APW,), jnp.float32),
        pltpu.VMEM((APW,), jnp.float32),
        pltpu.VMEM((APW,), jnp.float32),
        pltpu.VMEM((BROWS, 128), jnp.int32),
        pltpu.VMEM((APW,), jnp.float32),
        pltpu.VMEM((APW,), jnp.float32),
        pltpu.VMEM_SHARED((NM_PAD,), jnp.float32),
    ],
)
def _mol_kernel(part_hbm, q_hbm, mol_hbm, zeros_hbm, volt_hbm, bins_hbm,
                p0_v, p1_v, q_v, mol_v, v_v, c_v, bins_sh):
    cid = lax.axis_index("c")
    sid = lax.axis_index("s")
    wid = cid * NS + sid
    base = wid * APW

    @pl.when(sid == 0)
    def _():
        pltpu.sync_copy(zeros_hbm, bins_sh)

    pltpu.sync_copy(part_hbm.at[pl.ds(base, APW)], p0_v)
    pltpu.sync_copy(part_hbm.at[pl.ds(NA_PAD + base, APW)], p1_v)
    pltpu.sync_copy(q_hbm.at[pl.ds(base, APW)], q_v)
    pltpu.sync_copy(mol_hbm.at[pl.ds(wid * BROWS, BROWS), :], mol_v)
    plsc.subcore_barrier()

    def inner(i, c):
        s = pl.ds(i * L, L)
        v = p0_v[s] + p1_v[s]
        v_v[s] = v
        c_v[s] = (0.5 * v) * q_v[s]
        return c

    lax.fori_loop(0, APW // L, inner, 0)

    pltpu.sync_copy(v_v, volt_hbm.at[pl.ds(base, APW)])
    for j in range(BROWS):
        pltpu.sync_copy(c_v.at[pl.ds(j * 128, 128)],
                        bins_sh.at[mol_v.at[j]], add=True)

    plsc.subcore_barrier()

    @pl.when(sid == 0)
    def _():
        pltpu.sync_copy(bins_sh, bins_hbm.at[pl.ds(cid * NM_PAD, NM_PAD)])


def kernel(charges, pair_dist, pair_first, pair_second, mol_index, n_molecules):
    q = charges.reshape(NA)
    padp = NP_PAD - NP
    dist_p = jnp.concatenate([pair_dist, jnp.ones((padp,), jnp.float32)])
    first_p = jnp.concatenate(
        [pair_first, jnp.full((padp,), NA, jnp.int32)]).reshape(NP_PAD // 128, 128)
    sec_p = jnp.concatenate([pair_second, jnp.zeros((padp,), jnp.int32)])
    zeros_acc = jnp.zeros((NA_PAD,), jnp.float32)

    part = _pairs_kernel(q, dist_p, first_p, sec_p, zeros_acc)

    pada = NA_PAD - NA
    q_pad = jnp.concatenate([q, jnp.zeros((pada,), jnp.float32)])
    mol_p = jnp.concatenate(
        [mol_index, jnp.full((pada,), NM_PAD - 1, jnp.int32)]).reshape(NA_PAD // 128, 128)
    zeros_bins = jnp.zeros((NM_PAD,), jnp.float32)

    volt, bins = _mol_kernel(part, q_pad, mol_p, zeros_bins)

    voltage_atom = volt[:NA].reshape(NA, 1)
    coulomb_molecules = (bins[:NM] + bins[NM_PAD:NM_PAD + NM]).reshape(NM, 1)
    return (coulomb_molecules, voltage_atom)


# speedup vs baseline: 1.9658x; 1.4504x over previous
import functools

import jax
import jax.numpy as jnp
from jax import lax
from jax.experimental import pallas as pl
from jax.experimental.pallas import tpu as pltpu
from jax.experimental.pallas import tpu_sc as plsc

F = 14.399645

NA = 100000
NP = 6400000
NM = 1000

NC, NS, L = 2, 16, 16
NW = NC * NS

NA_PAD = 131072
CHUNK = 1024
ROWS_PER_CHUNK = CHUNK // 128
NCH_TOTAL = NP // CHUNK
NCH_MAIN = 195
NCH_PIPE = 192
PPW = NCH_MAIN * CHUNK
NCH_EXTRA = NCH_TOTAL - NW * NCH_MAIN

NM_PAD = 1024
APW = NA_PAD // NW
BROWS = APW // 128

_mesh = plsc.VectorSubcoreMesh(core_axis_name="c", subcore_axis_name="s")


@functools.partial(
    pl.kernel,
    mesh=_mesh,
    out_type=jax.ShapeDtypeStruct((NC * NA_PAD,), jnp.float32),
    compiler_params=pltpu.CompilerParams(needs_layout_passes=False),
    scratch_types=[
        pltpu.VMEM((NA,), jnp.float32),
        pltpu.VMEM((CHUNK,), jnp.int32),
        pltpu.VMEM((CHUNK,), jnp.int32),
        pltpu.VMEM((CHUNK,), jnp.float32),
        pltpu.VMEM((CHUNK,), jnp.float32),
        pltpu.VMEM((ROWS_PER_CHUNK, 128), jnp.int32),
        pltpu.VMEM((ROWS_PER_CHUNK, 128), jnp.int32),
        pltpu.VMEM((ROWS_PER_CHUNK, 128), jnp.int32),
        pltpu.VMEM((ROWS_PER_CHUNK, 128), jnp.int32),
        pltpu.VMEM((CHUNK,), jnp.float32),
        pltpu.VMEM((CHUNK,), jnp.float32),
        pltpu.VMEM((CHUNK,), jnp.float32),
        pltpu.VMEM((CHUNK,), jnp.float32),
        pltpu.VMEM_SHARED((NA_PAD,), jnp.float32),
        pltpu.SemaphoreType.DMA,
        pltpu.SemaphoreType.DMA,
        pltpu.SemaphoreType.DMA,
        pltpu.SemaphoreType.DMA,
        pltpu.SemaphoreType.DMA,
        pltpu.SemaphoreType.DMA,
    ],
)
def _pairs_kernel(q_hbm, dist_hbm, first_hbm, sec_hbm, zeros_hbm, part_hbm,
                  ch_v, sec0_v, sec1_v, dist0_v, dist1_v,
                  first0_v, first1_v, first2_v, first3_v,
                  v0_v, v1_v, v2_v, v3_v, acc_sh,
                  isem0, isem1, ssem0, ssem1, ssem2, ssem3):
    cid = lax.axis_index("c")
    sid = lax.axis_index("s")
    wid = cid * NS + sid
    sec_v = (sec0_v, sec1_v)
    dist_v = (dist0_v, dist1_v)
    first_v = (first0_v, first1_v, first2_v, first3_v)
    v_v = (v0_v, v1_v, v2_v, v3_v)
    isems = (isem0, isem1)
    ssems = (ssem0, ssem1, ssem2, ssem3)

    pltpu.sync_copy(q_hbm, ch_v)

    @pl.when(sid == 0)
    def _():
        pltpu.sync_copy(zeros_hbm, acc_sh)

    plsc.subcore_barrier()

    def chunk_off(k):
        return wid * PPW + k * CHUNK

    def chunk_row(k):
        return wid * (PPW // 128) + k * ROWS_PER_CHUNK

    def extra_off():
        return (NW * NCH_MAIN + wid) * CHUNK

    def extra_row():
        return NW * NCH_MAIN * ROWS_PER_CHUNK + wid * ROWS_PER_CHUNK

    def start_inputs(off, row, b2, b4):
        pltpu.async_copy(sec_hbm.at[pl.ds(off, CHUNK)], sec_v[b2], isems[b2])
        pltpu.async_copy(dist_hbm.at[pl.ds(off, CHUNK)], dist_v[b2], isems[b2])
        pltpu.async_copy(first_hbm.at[pl.ds(row, ROWS_PER_CHUNK), :],
                         first_v[b4], isems[b2])

    def wait_inputs(off, row, b2, b4):
        pltpu.make_async_copy(sec_hbm.at[pl.ds(off, CHUNK)], sec_v[b2],
                              isems[b2]).wait()
        pltpu.make_async_copy(dist_hbm.at[pl.ds(off, CHUNK)], dist_v[b2],
                              isems[b2]).wait()
        pltpu.make_async_copy(first_hbm.at[pl.ds(row, ROWS_PER_CHUNK), :],
                              first_v[b4], isems[b2]).wait()

    def drain_scatter(b4):
        for j in range(ROWS_PER_CHUNK):
            pltpu.make_async_copy(v_v[b4].at[pl.ds(j * 128, 128)],
                                  acc_sh.at[first_v[b4].at[j]],
                                  ssems[b4]).wait()

    def compute(b2, b4):
        sv, dv, vv = sec_v[b2], dist_v[b2], v_v[b4]

        def inner(i, c):
            s = pl.ds(i * L, L)
            idx = sv[s]
            q = plsc.load_gather(ch_v, [idx])
            d = dv[s]
            vv[s] = (F * q) / d
            return c

        lax.fori_loop(0, CHUNK // L, inner, 0, unroll=4)

    def issue_scatter(b4):
        for j in range(ROWS_PER_CHUNK):
            pltpu.async_copy(v_v[b4].at[pl.ds(j * 128, 128)],
                             acc_sh.at[first_v[b4].at[j]], ssems[b4],
                             add=True)

    start_inputs(chunk_off(0), chunk_row(0), 0, 0)

    def pair_body(t, carry):
        for u in range(4):
            k = 4 * t + u
            b2, b4 = u % 2, u
            wait_inputs(chunk_off(k), chunk_row(k), b2, b4)

            @pl.when(k >= 3)
            def _():
                drain_scatter((u + 1) % 4)

            @pl.when(k + 1 < NCH_PIPE)
            def _():
                start_inputs(chunk_off(k + 1), chunk_row(k + 1),
                             (u + 1) % 2, (u + 1) % 4)

            compute(b2, b4)
            issue_scatter(b4)
        return carry

    lax.fori_loop(0, NCH_PIPE // 4, pair_body, 0)
    drain_scatter(1)
    drain_scatter(2)
    drain_scatter(3)

    for j, k in enumerate(range(NCH_PIPE, NCH_MAIN)):
        pltpu.sync_copy(sec_hbm.at[pl.ds(chunk_off(k), CHUNK)], sec_v[j % 2])
        pltpu.sync_copy(dist_hbm.at[pl.ds(chunk_off(k), CHUNK)], dist_v[j % 2])
        pltpu.sync_copy(first_hbm.at[pl.ds(chunk_row(k), ROWS_PER_CHUNK), :],
                        first_v[j])
        compute(j % 2, j)
        issue_scatter(j)

    @pl.when(wid < NCH_EXTRA)
    def _():
        pltpu.sync_copy(sec_hbm.at[pl.ds(extra_off(), CHUNK)], sec_v[0])
        pltpu.sync_copy(dist_hbm.at[pl.ds(extra_off(), CHUNK)], dist_v[0])
        pltpu.sync_copy(first_hbm.at[pl.ds(extra_row(), ROWS_PER_CHUNK), :],
                        first_v[3])
        compute(0, 3)
        issue_scatter(3)

    drain_scatter(0)
    drain_scatter(1)
    drain_scatter(2)

    @pl.when(wid < NCH_EXTRA)
    def _():
        drain_scatter(3)

    plsc.subcore_barrier()
    seg = NA_PAD // NS
    pltpu.sync_copy(acc_sh.at[pl.ds(sid * seg, seg)],
                    part_hbm.at[pl.ds(cid * NA_PAD + sid * seg, seg)])


@functools.partial(
    pl.kernel,
    mesh=_mesh,
    out_type=(
        jax.ShapeDtypeStruct((NA_PAD,), jnp.float32),
        jax.ShapeDtypeStruct((NC * NM_PAD,), jnp.float32),
    ),
    compiler_params=pltpu.CompilerParams(needs_layout_passes=False),
    scratch_types=[
        pltpu.VMEM((APW,), jnp.float32),
        pltpu.VMEM((APW,), jnp.float32),
        pltpu.VMEM((APW,), jnp.float32),
        pltpu.VMEM((BROWS, 128), jnp.int32),
        pltpu.VMEM((APW,), jnp.float32),
        pltpu.VMEM((APW,), jnp.float32),
        pltpu.VMEM_SHARED((NM_PAD,), jnp.float32),
    ],
)
def _mol_kernel(part_hbm, q_hbm, mol_hbm, zeros_hbm, volt_hbm, bins_hbm,
                p0_v, p1_v, q_v, mol_v, v_v, c_v, bins_sh):
    cid = lax.axis_index("c")
    sid = lax.axis_index("s")
    wid = cid * NS + sid
    base = wid * APW

    @pl.when(sid == 0)
    def _():
        pltpu.sync_copy(zeros_hbm, bins_sh)

    pltpu.sync_copy(part_hbm.at[pl.ds(base, APW)], p0_v)
    pltpu.sync_copy(part_hbm.at[pl.ds(NA_PAD + base, APW)], p1_v)
    pltpu.sync_copy(q_hbm.at[pl.ds(base, APW)], q_v)
    pltpu.sync_copy(mol_hbm.at[pl.ds(wid * BROWS, BROWS), :], mol_v)
    plsc.subcore_barrier()

    def inner(i, c):
        s = pl.ds(i * L, L)
        v = p0_v[s] + p1_v[s]
        v_v[s] = v
        c_v[s] = (0.5 * v) * q_v[s]
        return c

    lax.fori_loop(0, APW // L, inner, 0)

    pltpu.sync_copy(v_v, volt_hbm.at[pl.ds(base, APW)])
    for j in range(BROWS):
        pltpu.sync_copy(c_v.at[pl.ds(j * 128, 128)],
                        bins_sh.at[mol_v.at[j]], add=True)

    plsc.subcore_barrier()

    @pl.when(sid == 0)
    def _():
        pltpu.sync_copy(bins_sh, bins_hbm.at[pl.ds(cid * NM_PAD, NM_PAD)])


def kernel(charges, pair_dist, pair_first, pair_second, mol_index, n_molecules):
    q = charges.reshape(NA)
    first_2d = pair_first.reshape(NP // 128, 128)
    zeros_acc = jnp.zeros((NA_PAD,), jnp.float32)

    part = _pairs_kernel(q, pair_dist, first_2d, pair_second, zeros_acc)

    pada = NA_PAD - NA
    q_pad = jnp.concatenate([q, jnp.zeros((pada,), jnp.float32)])
    mol_p = jnp.concatenate(
        [mol_index, jnp.full((pada,), NM_PAD - 1, jnp.int32)]).reshape(NA_PAD // 128, 128)
    zeros_bins = jnp.zeros((NM_PAD,), jnp.float32)

    volt, bins = _mol_kernel(part, q_pad, mol_p, zeros_bins)

    voltage_atom = volt[:NA].reshape(NA, 1)
    coulomb_molecules = (bins[:NM] + bins[NM_PAD:NM_PAD + NM]).reshape(NM, 1)
    return (coulomb_molecules, voltage_atom)
